# Initial kernel scaffold; baseline (speedup 1.0000x reference)
import jax
import jax.numpy as jnp
from jax import lax
from jax.experimental import pallas as pl
from jax.experimental.pallas import tpu as pltpu
from jax.experimental.pallas import tpu_sc as plsc

N = 10000
E = 320000
H = 2
C = 64
HC = H * C
OUT2 = 64 // 3

NCORES = 2
NSUB = 16
NW = NCORES * NSUB
EPT = 10240
EPAD = EPT * NW
CHW = 128
NCH = EPT // CHW
ROWS_PER = 1000

_SLOPE = 0.2


def _leaky(v):
    return jnp.where(v >= 0, v, _SLOPE * v)


def _mean_body(a0_ref, a1_ref, o_ref):
    o_ref[0, 0] = jnp.sum(a0_ref[...])
    o_ref[0, 1] = jnp.sum(a1_ref[...])


def _attr_sums(a0m, a1m):
    return pl.pallas_call(
        _mean_body,
        out_shape=jax.ShapeDtypeStruct((1, 2), jnp.float32),
        in_specs=[
            pl.BlockSpec(a0m.shape, lambda: (0, 0)),
            pl.BlockSpec(a1m.shape, lambda: (0, 0)),
        ],
        out_specs=pl.BlockSpec(memory_space=pltpu.SMEM),
    )(a0m, a1m)


def _gather_body(x_hbm, src_hbm, dst_hbm, sj_hbm, si_hbm, x_v, idx_v, out_v):
    c = lax.axis_index("c")
    s = lax.axis_index("s")
    wid = s * NCORES + c
    base = wid * EPT
    pltpu.sync_copy(x_hbm, x_v)
    for idx_hbm, o_hbm in ((src_hbm, sj_hbm), (dst_hbm, si_hbm)):
        pltpu.sync_copy(idx_hbm.at[pl.ds(base, EPT)], idx_v)

        def body(i, _):
            off = pl.multiple_of(i * 16, 16)
            ids = idx_v[pl.ds(off, 16)]
            out_v[pl.ds(off, 16)] = plsc.load_gather(x_v, [ids])
            return 0

        lax.fori_loop(0, EPT // 16, body, 0)
        pltpu.sync_copy(out_v, o_hbm.at[pl.ds(base, EPT)])


def _gather(x_flat, src_p, dst_p):
    mesh = plsc.VectorSubcoreMesh(
        core_axis_name="c", subcore_axis_name="s",
        num_cores=NCORES, num_subcores=NSUB)
    f = pl.kernel(
        _gather_body,
        out_type=(
            jax.ShapeDtypeStruct((EPAD,), jnp.float32),
            jax.ShapeDtypeStruct((EPAD,), jnp.float32),
        ),
        mesh=mesh,
        scratch_types=[
            pltpu.VMEM((N,), jnp.float32),
            pltpu.VMEM((EPT,), jnp.int32),
            pltpu.VMEM((EPT,), jnp.float32),
        ],
    )
    return f(x_flat, src_p, dst_p)


BE = 2048


def _edge_body(m_ref, w_ref, sj_ref, si_ref, a0_ref, a1_ref, o_ref):
    inv_e = 1.0 / E
    m0 = m_ref[0, 0] * inv_e
    m1 = m_ref[0, 1] * inv_e
    w = w_ref[...]
    wl = w[0:1, :]
    wr = w[1:2, :]
    we0 = w[2:3, :]
    we1 = w[3:4, :]
    blr = w[4:5, :]
    attv = w[5:6, :]
    wlr = w[6:7, :]

    sj = sj_ref[...]
    si = si_ref[...]
    a0 = a0_ref[...]
    a1 = a1_ref[...]

    pre = sj * wl + si * wr + (a0 * we0 + a1 * we1 + blr)
    za = _leaky(pre) * attv
    al0 = jnp.sum(za[:, :C], axis=1, keepdims=True)
    al1 = jnp.sum(za[:, C:], axis=1, keepdims=True)

    pre_s = si * wlr + (m0 * we0 + m1 * we1 + blr)
    zsa = _leaky(pre_s) * attv
    as0 = jnp.sum(zsa[:, :C], axis=1, keepdims=True)
    as1 = jnp.sum(zsa[:, C:], axis=1, keepdims=True)

    p0 = jnp.exp(al0 - as0)
    p1 = jnp.exp(al1 - as1)
    eid = pl.program_id(0) * BE + lax.broadcasted_iota(jnp.int32, (BE, 1), 0)
    valid = eid < E
    p0 = jnp.where(valid, p0, 0.0)
    p1 = jnp.where(valid, p1, 0.0)
    o_ref[...] = jnp.concatenate([p0, p1, p0 * sj, p1 * sj], axis=1)


def _edge_vals(msum, wmat, sj, si, a0p, a1p):
    col = pl.BlockSpec((BE, 1), lambda i: (i, 0))
    return pl.pallas_call(
        _edge_body,
        grid=(EPAD // BE,),
        out_shape=jax.ShapeDtypeStruct((EPAD, 4), jnp.float32),
        in_specs=[
            pl.BlockSpec(memory_space=pltpu.SMEM),
            pl.BlockSpec((8, HC), lambda i: (0, 0)),
            col, col, col, col,
        ],
        out_specs=pl.BlockSpec((BE, 4), lambda i: (i, 0)),
    )(msum, wmat, sj, si, a0p, a1p)


def _scatter_body(z_hbm, dst_hbm, vals_hbm, out_hbm, idx_v, vals_v, acc):
    c = lax.axis_index("c")
    s = lax.axis_index("s")
    wid = s * NCORES + c

    @pl.when(s < N // ROWS_PER)
    def _():
        rb = s * ROWS_PER
        pltpu.sync_copy(z_hbm.at[pl.ds(rb, ROWS_PER), :],
                        acc.at[pl.ds(rb, ROWS_PER), :])

    plsc.subcore_barrier()

    pltpu.sync_copy(dst_hbm.at[pl.ds(wid * NCH, NCH), :], idx_v)
    pltpu.sync_copy(vals_hbm.at[pl.ds(wid * NCH, NCH), :, :], vals_v)

    def body(j, _):
        pltpu.sync_copy(vals_v.at[j], acc.at[idx_v.at[j]], add=True)
        return 0

    lax.fori_loop(0, NCH, body, 0)

    plsc.subcore_barrier()

    @pl.when(s < N // ROWS_PER)
    def _():
        rb = s * ROWS_PER
        pltpu.sync_copy(acc.at[pl.ds(rb, ROWS_PER), :],
                        out_hbm.at[c, pl.ds(rb, ROWS_PER), :])


def _scatter(zeros_n4, dst2, vals3):
    mesh = plsc.VectorSubcoreMesh(
        core_axis_name="c", subcore_axis_name="s",
        num_cores=NCORES, num_subcores=NSUB)
    f = pl.kernel(
        _scatter_body,
        out_type=jax.ShapeDtypeStruct((NCORES, N, 4), jnp.float32),
        mesh=mesh,
        scratch_types=[
            pltpu.VMEM((NCH, CHW), jnp.int32),
            pltpu.VMEM((NCH, CHW, 4), jnp.float32),
            pltpu.VMEM_SHARED((N, 4), jnp.float32),
        ],
    )
    return f(zeros_n4, dst2, vals3)


def _final_body(acc_ref, x_ref, u1_ref, cv_ref, o_ref):
    t = acc_ref[0] + acc_ref[1]
    den = t[:, 0:2] + 1.0
    num = t[:, 2:4] + x_ref[...]
    s1 = num / den
    u = u1_ref[...]
    o_ref[...] = s1[:, 0:1] * u[0:1, :] + s1[:, 1:2] * u[1:2, :] + cv_ref[...]


def _final(acc, x, u1, cvec):
    return pl.pallas_call(
        _final_body,
        out_shape=jax.ShapeDtypeStruct((N, OUT2), jnp.float32),
        in_specs=[
            pl.BlockSpec((NCORES, N, 4), lambda: (0, 0, 0)),
            pl.BlockSpec((N, 1), lambda: (0, 0)),
            pl.BlockSpec((2, OUT2), lambda: (0, 0)),
            pl.BlockSpec((1, OUT2), lambda: (0, 0)),
        ],
        out_specs=pl.BlockSpec((N, OUT2), lambda: (0, 0)),
    )(acc, x, u1, cvec)


def kernel(x, edge_index, edge_attr, W_l, b_l, W_r, b_r, W_e, att, bias, W2, b2):
    src = edge_index[0].astype(jnp.int32)
    dst = edge_index[1].astype(jnp.int32)
    pad = EPAD - E
    src_p = jnp.concatenate([src, jnp.zeros((pad,), jnp.int32)])
    dst_p = jnp.concatenate([dst, jnp.zeros((pad,), jnp.int32)])
    a0 = edge_attr[:, 0]
    a1 = edge_attr[:, 1]
    zf = jnp.zeros((pad,), jnp.float32)
    a0p = jnp.concatenate([a0, zf])[:, None]
    a1p = jnp.concatenate([a1, zf])[:, None]
    a0m = a0.reshape(E // 256, 256)
    a1m = a1.reshape(E // 256, 256)

    wl = W_l[0]
    wr = W_r[0]
    blr = b_l + b_r
    attv = att.reshape(HC)
    wmat = jnp.stack([wl, wr, W_e[0], W_e[1], blr, attv, wl + wr,
                      jnp.zeros_like(wl)])
    u1 = jnp.einsum("hc,hco->ho", W_l.reshape(H, C), W2.reshape(H, C, OUT2))
    cvec = ((b_l + bias) @ W2 + b2)[None, :]

    msum = _attr_sums(a0m, a1m)
    x_flat = x.reshape(N)
    sj, si = _gather(x_flat, src_p, dst_p)
    vals = _edge_vals(msum, wmat, sj[:, None], si[:, None], a0p, a1p)
    acc = _scatter(jnp.zeros((N, 4), jnp.float32),
                   dst_p.reshape(NW * NCH, CHW),
                   vals.reshape(NW * NCH, CHW, 4))
    d = _final(acc, x, u1, cvec)
    return d.reshape(1, N * OUT2)


# speedup vs baseline: 36.1876x; 36.1876x over previous
import jax
import jax.numpy as jnp
from jax import lax
from jax.experimental import pallas as pl
from jax.experimental.pallas import tpu as pltpu
from jax.experimental.pallas import tpu_sc as plsc

N = 10000
E = 320000
H = 2
C = 64
HC = H * C
OUT2 = 64 // 3

NCORES = 2
NSUB = 16
NW = NCORES * NSUB
EPT = 10240
EPAD = EPT * NW
CHW = 128
NCH = EPT // CHW
ROWS_PER = 1000

_SLOPE = 0.2


def _leaky(v):
    return jnp.where(v >= 0, v, _SLOPE * v)


def _mean_body(a0_ref, a1_ref, o_ref):
    o_ref[0, 0] = jnp.sum(a0_ref[...])
    o_ref[0, 1] = jnp.sum(a1_ref[...])


def _attr_sums(a0m, a1m):
    return pl.pallas_call(
        _mean_body,
        out_shape=jax.ShapeDtypeStruct((1, 2), jnp.float32),
        in_specs=[
            pl.BlockSpec(a0m.shape, lambda: (0, 0)),
            pl.BlockSpec(a1m.shape, lambda: (0, 0)),
        ],
        out_specs=pl.BlockSpec(memory_space=pltpu.SMEM),
    )(a0m, a1m)


def _gather_body(x_hbm, src_hbm, dst_hbm, sj_hbm, si_hbm, x_v, idx_v, out_v):
    c = lax.axis_index("c")
    s = lax.axis_index("s")
    wid = s * NCORES + c
    base = wid * EPT
    pltpu.sync_copy(x_hbm, x_v)
    for idx_hbm, o_hbm in ((src_hbm, sj_hbm), (dst_hbm, si_hbm)):
        pltpu.sync_copy(idx_hbm.at[pl.ds(base, EPT)], idx_v)

        def body(i, _):
            off = pl.multiple_of(i * 16, 16)
            ids = idx_v[pl.ds(off, 16)]
            out_v[pl.ds(off, 16)] = plsc.load_gather(x_v, [ids])
            return 0

        lax.fori_loop(0, EPT // 16, body, 0)
        pltpu.sync_copy(out_v, o_hbm.at[pl.ds(base, EPT)])


def _gather(x_flat, src_p, dst_p):
    mesh = plsc.VectorSubcoreMesh(
        core_axis_name="c", subcore_axis_name="s",
        num_cores=NCORES, num_subcores=NSUB)
    f = pl.kernel(
        _gather_body,
        out_type=(
            jax.ShapeDtypeStruct((EPAD,), jnp.float32),
            jax.ShapeDtypeStruct((EPAD,), jnp.float32),
        ),
        mesh=mesh,
        compiler_params=pltpu.CompilerParams(needs_layout_passes=False, use_tc_tiling_on_sc=False),
        scratch_types=[
            pltpu.VMEM((N,), jnp.float32),
            pltpu.VMEM((EPT,), jnp.int32),
            pltpu.VMEM((EPT,), jnp.float32),
        ],
    )
    return f(x_flat, src_p, dst_p)


BE = 2048


def _edge_body(m_ref, w_ref, sj_ref, si_ref, a0_ref, a1_ref,
               o0_ref, o1_ref, o2_ref, o3_ref):
    inv_e = 1.0 / E
    m0 = m_ref[0, 0] * inv_e
    m1 = m_ref[0, 1] * inv_e
    w = w_ref[...]
    wl = w[0:1, :]
    wr = w[1:2, :]
    we0 = w[2:3, :]
    we1 = w[3:4, :]
    blr = w[4:5, :]
    attv = w[5:6, :]
    wlr = w[6:7, :]

    sj = sj_ref[...]
    si = si_ref[...]
    a0 = a0_ref[...]
    a1 = a1_ref[...]

    pre = sj * wl + si * wr + (a0 * we0 + a1 * we1 + blr)
    za = _leaky(pre) * attv
    al0 = jnp.sum(za[:, :C], axis=1, keepdims=True)
    al1 = jnp.sum(za[:, C:], axis=1, keepdims=True)

    pre_s = si * wlr + (m0 * we0 + m1 * we1 + blr)
    zsa = _leaky(pre_s) * attv
    as0 = jnp.sum(zsa[:, :C], axis=1, keepdims=True)
    as1 = jnp.sum(zsa[:, C:], axis=1, keepdims=True)

    p0 = jnp.exp(al0 - as0)
    p1 = jnp.exp(al1 - as1)
    eid = pl.program_id(0) * BE + lax.broadcasted_iota(jnp.int32, (BE, 1), 0)
    valid = eid < E
    p0 = jnp.where(valid, p0, 0.0)
    p1 = jnp.where(valid, p1, 0.0)
    o0_ref[...] = p0
    o1_ref[...] = p1
    o2_ref[...] = p0 * sj
    o3_ref[...] = p1 * sj


def _edge_vals(msum, wmat, sj, si, a0p, a1p):
    col = pl.BlockSpec((BE, 1), lambda i: (i, 0))
    return pl.pallas_call(
        _edge_body,
        grid=(EPAD // BE,),
        out_shape=[jax.ShapeDtypeStruct((EPAD, 1), jnp.float32)] * 4,
        in_specs=[
            pl.BlockSpec(memory_space=pltpu.SMEM),
            pl.BlockSpec((8, HC), lambda i: (0, 0)),
            col, col, col, col,
        ],
        out_specs=[pl.BlockSpec((BE, 1), lambda i: (i, 0))] * 4,
    )(msum, wmat, sj, si, a0p, a1p)


def _scatter_body(dst_hbm, p0_hbm, p1_hbm, q0_hbm, q1_hbm, out_hbm,
                  idx_v, v0_v, v1_v, v2_v, v3_v, acc_v):
    c = lax.axis_index("c")
    s = lax.axis_index("s")
    wid = s * NCORES + c
    base = wid * EPT

    def zero(i, _):
        off = pl.multiple_of(i * 16, 16)
        acc_v[pl.ds(off, 16)] = jnp.zeros((16,), jnp.float32)
        return 0

    lax.fori_loop(0, (4 * N) // 16, zero, 0)

    pltpu.sync_copy(dst_hbm.at[pl.ds(base, EPT)], idx_v)
    pltpu.sync_copy(p0_hbm.at[pl.ds(base, EPT)], v0_v)
    pltpu.sync_copy(p1_hbm.at[pl.ds(base, EPT)], v1_v)
    pltpu.sync_copy(q0_hbm.at[pl.ds(base, EPT)], v2_v)
    pltpu.sync_copy(q1_hbm.at[pl.ds(base, EPT)], v3_v)

    def scat(i, _):
        off = pl.multiple_of(i * 16, 16)
        ids = idx_v[pl.ds(off, 16)]
        plsc.addupdate_scatter(acc_v, [ids], v0_v[pl.ds(off, 16)])
        plsc.addupdate_scatter(acc_v, [ids + N], v1_v[pl.ds(off, 16)])
        plsc.addupdate_scatter(acc_v, [ids + 2 * N], v2_v[pl.ds(off, 16)])
        plsc.addupdate_scatter(acc_v, [ids + 3 * N], v3_v[pl.ds(off, 16)])
        return 0

    lax.fori_loop(0, EPT // 16, scat, 0)

    pltpu.sync_copy(acc_v, out_hbm.at[wid])


def _scatter(dst_p, p0, p1, q0, q1):
    mesh = plsc.VectorSubcoreMesh(
        core_axis_name="c", subcore_axis_name="s",
        num_cores=NCORES, num_subcores=NSUB)
    f = pl.kernel(
        _scatter_body,
        out_type=jax.ShapeDtypeStruct((NW, 4 * N), jnp.float32),
        mesh=mesh,
        compiler_params=pltpu.CompilerParams(
            needs_layout_passes=False, use_tc_tiling_on_sc=False),
        scratch_types=[
            pltpu.VMEM((EPT,), jnp.int32),
            pltpu.VMEM((EPT,), jnp.float32),
            pltpu.VMEM((EPT,), jnp.float32),
            pltpu.VMEM((EPT,), jnp.float32),
            pltpu.VMEM((EPT,), jnp.float32),
            pltpu.VMEM((4 * N,), jnp.float32),
        ],
    )
    return f(dst_p, p0, p1, q0, q1)


def _final_body(acc_ref, x_ref, u1_ref, cv_ref, o_ref):
    a = acc_ref[0]
    for t in range(1, NW):
        a = a + acc_ref[t]
    den = a[0:2, :] + 1.0
    num = a[2:4, :] + x_ref[...]
    s1 = num / den
    u = u1_ref[...]
    d = jax.lax.dot_general(s1, u, (((0,), (0,)), ((), ())),
                            preferred_element_type=jnp.float32)
    o_ref[...] = d + cv_ref[...]


def _final(acc, xrow, u1, cvec):
    return pl.pallas_call(
        _final_body,
        out_shape=jax.ShapeDtypeStruct((N, OUT2), jnp.float32),
        in_specs=[
            pl.BlockSpec((NW, 4, N), lambda: (0, 0, 0)),
            pl.BlockSpec((1, N), lambda: (0, 0)),
            pl.BlockSpec((2, OUT2), lambda: (0, 0)),
            pl.BlockSpec((1, OUT2), lambda: (0, 0)),
        ],
        out_specs=pl.BlockSpec((N, OUT2), lambda: (0, 0)),
    )(acc, xrow, u1, cvec)


def kernel(x, edge_index, edge_attr, W_l, b_l, W_r, b_r, W_e, att, bias, W2, b2):
    src = edge_index[0].astype(jnp.int32)
    dst = edge_index[1].astype(jnp.int32)
    pad = EPAD - E
    src_p = jnp.concatenate([src, jnp.zeros((pad,), jnp.int32)])
    dst_p = jnp.concatenate([dst, jnp.zeros((pad,), jnp.int32)])
    a0 = edge_attr[:, 0]
    a1 = edge_attr[:, 1]
    zf = jnp.zeros((pad,), jnp.float32)
    a0p = jnp.concatenate([a0, zf])[:, None]
    a1p = jnp.concatenate([a1, zf])[:, None]
    a0m = a0.reshape(E // 256, 256)
    a1m = a1.reshape(E // 256, 256)

    wl = W_l[0]
    wr = W_r[0]
    blr = b_l + b_r
    attv = att.reshape(HC)
    wmat = jnp.stack([wl, wr, W_e[0], W_e[1], blr, attv, wl + wr,
                      jnp.zeros_like(wl)])
    u1 = jnp.einsum("hc,hco->ho", W_l.reshape(H, C), W2.reshape(H, C, OUT2))
    cvec = ((b_l + bias) @ W2 + b2)[None, :]

    msum = _attr_sums(a0m, a1m)
    x_flat = x.reshape(N)
    sj, si = _gather(x_flat, src_p, dst_p)
    p0, p1, q0, q1 = _edge_vals(msum, wmat, sj[:, None], si[:, None],
                                a0p, a1p)
    acc = _scatter(dst_p, p0.reshape(EPAD), p1.reshape(EPAD),
                   q0.reshape(EPAD), q1.reshape(EPAD))
    d = _final(acc.reshape(NW, 4, N), x.reshape(1, N), u1, cvec)
    return d.reshape(1, N * OUT2)
